# Initial kernel scaffold; baseline (speedup 1.0000x reference)
import jax
import jax.numpy as jnp
from jax.experimental import pallas as pl


def kernel(input_tensor, W1, b1, W2, b2, Wg, bg):
    raise NotImplementedError("write your pallas kernel here")


# speedup vs baseline: 6.3053x; 6.3053x over previous
import functools

import jax
import jax.numpy as jnp
from jax.experimental import pallas as pl
from jax.experimental.pallas import tpu as pltpu

_B, _L, _H = 2, 2048, 1024
_A, _R, _K = 8, 128, 2
_N = _B * _L
_BLK = 256


def _fused_body(x_ref, wgt_ref, bg_ref, w1t_ref, b1_ref, w2f_ref, b2_ref, out_ref):
    x = x_ref[...]
    logits = jnp.dot(x, wgt_ref[...], preferred_element_type=jnp.float32)
    logits = logits + bg_ref[...]
    iota_a = jax.lax.broadcasted_iota(jnp.int32, (1, _A), 1)
    sel = []
    for a in range(_A):
        la = logits[:, a : a + 1]
        gt = (logits > la).astype(jnp.int32)
        eq = jnp.logical_and(logits == la, iota_a < a).astype(jnp.int32)
        rank = jnp.sum(gt + eq, axis=1, keepdims=True)
        sel.append((rank < _K).astype(jnp.float32))
    h = jnp.dot(x, w1t_ref[...], preferred_element_type=jnp.float32)
    h = h + b1_ref[...]
    h = 0.5 * h * (1.0 + jax.lax.erf(h * 0.7071067811865476))
    hm = jnp.concatenate(
        [h[:, a * _R : (a + 1) * _R] * sel[a] for a in range(_A)], axis=1
    )
    y = jnp.dot(hm, w2f_ref[...], preferred_element_type=jnp.float32)
    bias = sel[0] * b2_ref[0][None, :]
    for a in range(1, _A):
        bias = bias + sel[a] * b2_ref[a][None, :]
    out_ref[...] = (y + bias) * (2.0 / _K)


@jax.jit
def _fused(x2d, wgt, bg2, w1t, b12, w2f, b2):
    grid = (_N // _BLK,)
    return pl.pallas_call(
        _fused_body,
        grid=grid,
        in_specs=[
            pl.BlockSpec((_BLK, _H), lambda i: (i, 0)),
            pl.BlockSpec((_H, _A), lambda i: (0, 0)),
            pl.BlockSpec((1, _A), lambda i: (0, 0)),
            pl.BlockSpec((_H, _A * _R), lambda i: (0, 0)),
            pl.BlockSpec((1, _A * _R), lambda i: (0, 0)),
            pl.BlockSpec((_A * _R, _H), lambda i: (0, 0)),
            pl.BlockSpec((_A, _H), lambda i: (0, 0)),
        ],
        out_specs=pl.BlockSpec((_BLK, _H), lambda i: (i, 0)),
        out_shape=jax.ShapeDtypeStruct((_N, _H), jnp.float32),
        compiler_params=pltpu.CompilerParams(
            dimension_semantics=("arbitrary",),
        ),
    )(x2d, wgt, bg2, w1t, b12, w2f, b2)


def kernel(input_tensor, W1, b1, W2, b2, Wg, bg):
    x2d = input_tensor.reshape(_N, _H)
    wgt = Wg.T
    bg2 = bg.reshape(1, _A)
    w1t = W1.T
    b12 = b1.reshape(1, _A * _R)
    w2f = W2.transpose(0, 2, 1).reshape(_A * _R, _H)
    y = _fused(x2d, wgt, bg2, w1t, b12, w2f, b2)
    return y.reshape(_B, _L, _H)


# speedup vs baseline: 6.4830x; 1.0282x over previous
import functools

import jax
import jax.numpy as jnp
from jax.experimental import pallas as pl
from jax.experimental.pallas import tpu as pltpu

_B, _L, _H = 2, 2048, 1024
_A, _R, _K = 8, 128, 2
_N = _B * _L
_BLK = 256


def _fused_body(x_ref, wgt_ref, bg_ref, w1t_ref, b1_ref, w2f_ref, b2_ref, out_ref):
    x = x_ref[...]
    logits = jnp.dot(x, wgt_ref[...], preferred_element_type=jnp.float32)
    logits = logits + bg_ref[...]
    iota_a = jax.lax.broadcasted_iota(jnp.int32, (1, _A), 1)
    sel = []
    for a in range(_A):
        la = logits[:, a : a + 1]
        gt = (logits > la).astype(jnp.int32)
        eq = jnp.logical_and(logits == la, iota_a < a).astype(jnp.int32)
        rank = jnp.sum(gt + eq, axis=1, keepdims=True)
        sel.append((rank < _K).astype(jnp.float32))
    xb = x.astype(jnp.bfloat16)
    h = jnp.dot(xb, w1t_ref[...], preferred_element_type=jnp.float32)
    h = h + b1_ref[...]
    h = 0.5 * h * (1.0 + jax.lax.erf(h * 0.7071067811865476))
    hm = jnp.concatenate(
        [h[:, a * _R : (a + 1) * _R] * sel[a] for a in range(_A)], axis=1
    ).astype(jnp.bfloat16)
    y = jnp.dot(hm, w2f_ref[...], preferred_element_type=jnp.float32)
    bias = sel[0] * b2_ref[0][None, :]
    for a in range(1, _A):
        bias = bias + sel[a] * b2_ref[a][None, :]
    out_ref[...] = (y + bias) * (2.0 / _K)


@jax.jit
def _fused(x2d, wgt, bg2, w1t, b12, w2f, b2):
    grid = (_N // _BLK,)
    return pl.pallas_call(
        _fused_body,
        grid=grid,
        in_specs=[
            pl.BlockSpec((_BLK, _H), lambda i: (i, 0)),
            pl.BlockSpec((_H, _A), lambda i: (0, 0)),
            pl.BlockSpec((1, _A), lambda i: (0, 0)),
            pl.BlockSpec((_H, _A * _R), lambda i: (0, 0)),
            pl.BlockSpec((1, _A * _R), lambda i: (0, 0)),
            pl.BlockSpec((_A * _R, _H), lambda i: (0, 0)),
            pl.BlockSpec((_A, _H), lambda i: (0, 0)),
        ],
        out_specs=pl.BlockSpec((_BLK, _H), lambda i: (i, 0)),
        out_shape=jax.ShapeDtypeStruct((_N, _H), jnp.float32),
        compiler_params=pltpu.CompilerParams(
            dimension_semantics=("arbitrary",),
        ),
    )(x2d, wgt, bg2, w1t, b12, w2f, b2)


def kernel(input_tensor, W1, b1, W2, b2, Wg, bg):
    x2d = input_tensor.reshape(_N, _H)
    wgt = Wg.T
    bg2 = bg.reshape(1, _A)
    w1t = W1.T.astype(jnp.bfloat16)
    b12 = b1.reshape(1, _A * _R)
    w2f = W2.transpose(0, 2, 1).reshape(_A * _R, _H).astype(jnp.bfloat16)
    y = _fused(x2d, wgt, bg2, w1t, b12, w2f, b2)
    return y.reshape(_B, _L, _H)


# speedup vs baseline: 7.7515x; 1.1957x over previous
import functools

import jax
import jax.numpy as jnp
from jax.experimental import pallas as pl
from jax.experimental.pallas import tpu as pltpu

_B, _L, _H = 2, 2048, 1024
_A, _R, _K = 8, 128, 2
_N = _B * _L
_BLK = 256


def _fused_body(x_ref, wgt_ref, bg_ref, w1t_ref, b1_ref, w2f_ref, b2_ref, out_ref):
    x = x_ref[...]
    logits = jnp.dot(x, wgt_ref[...], preferred_element_type=jnp.float32)
    logits = logits + bg_ref[...]
    iota_a = jax.lax.broadcasted_iota(jnp.int32, (_BLK, _A), 1)
    m1 = jnp.max(logits, axis=1, keepdims=True)
    i1 = jnp.min(jnp.where(logits == m1, iota_a, _A), axis=1, keepdims=True)
    l2 = jnp.where(iota_a == i1, -jnp.inf, logits)
    m2 = jnp.max(l2, axis=1, keepdims=True)
    i2 = jnp.min(jnp.where(l2 == m2, iota_a, _A), axis=1, keepdims=True)
    selmat = jnp.logical_or(iota_a == i1, iota_a == i2).astype(jnp.float32)
    sel = [selmat[:, a : a + 1] for a in range(_A)]
    xb = x.astype(jnp.bfloat16)
    h = jnp.dot(xb, w1t_ref[...], preferred_element_type=jnp.float32)
    h = h + b1_ref[...]
    h = 0.5 * h * (1.0 + jax.lax.erf(h * 0.7071067811865476))
    hm = jnp.concatenate(
        [h[:, a * _R : (a + 1) * _R] * sel[a] for a in range(_A)], axis=1
    ).astype(jnp.bfloat16)
    y = jnp.dot(hm, w2f_ref[...], preferred_element_type=jnp.float32)
    bias = jnp.dot(selmat, b2_ref[...], preferred_element_type=jnp.float32)
    out_ref[...] = y + bias


@jax.jit
def _fused(x2d, wgt, bg2, w1t, b12, w2f, b2):
    grid = (_N // _BLK,)
    return pl.pallas_call(
        _fused_body,
        grid=grid,
        in_specs=[
            pl.BlockSpec((_BLK, _H), lambda i: (i, 0)),
            pl.BlockSpec((_H, _A), lambda i: (0, 0)),
            pl.BlockSpec((1, _A), lambda i: (0, 0)),
            pl.BlockSpec((_H, _A * _R), lambda i: (0, 0)),
            pl.BlockSpec((1, _A * _R), lambda i: (0, 0)),
            pl.BlockSpec((_A * _R, _H), lambda i: (0, 0)),
            pl.BlockSpec((_A, _H), lambda i: (0, 0)),
        ],
        out_specs=pl.BlockSpec((_BLK, _H), lambda i: (i, 0)),
        out_shape=jax.ShapeDtypeStruct((_N, _H), jnp.float32),
        compiler_params=pltpu.CompilerParams(
            dimension_semantics=("arbitrary",),
        ),
    )(x2d, wgt, bg2, w1t, b12, w2f, b2)


def kernel(input_tensor, W1, b1, W2, b2, Wg, bg):
    x2d = input_tensor.reshape(_N, _H)
    wgt = Wg.T
    bg2 = bg.reshape(1, _A)
    w1t = W1.T.astype(jnp.bfloat16)
    b12 = b1.reshape(1, _A * _R)
    scale = 2.0 / _K
    w2f = (W2.transpose(0, 2, 1).reshape(_A * _R, _H) * scale).astype(jnp.bfloat16)
    y = _fused(x2d, wgt, bg2, w1t, b12, w2f, b2 * scale)
    return y.reshape(_B, _L, _H)


# speedup vs baseline: 8.3516x; 1.0774x over previous
import functools

import jax
import jax.numpy as jnp
from jax.experimental import pallas as pl
from jax.experimental.pallas import tpu as pltpu

_B, _L, _H = 2, 2048, 1024
_A, _R, _K = 8, 128, 2
_N = _B * _L
_BLK = 512


def _fused_body(x_ref, wgt_ref, bg_ref, w1t_ref, b1_ref, w2f_ref, b2_ref, out_ref):
    x = x_ref[...]
    logits = jnp.dot(x, wgt_ref[...], preferred_element_type=jnp.float32)
    logits = logits + bg_ref[...]
    iota_a = jax.lax.broadcasted_iota(jnp.int32, (_BLK, _A), 1)
    m1 = jnp.max(logits, axis=1, keepdims=True)
    i1 = jnp.min(jnp.where(logits == m1, iota_a, _A), axis=1, keepdims=True)
    l2 = jnp.where(iota_a == i1, -jnp.inf, logits)
    m2 = jnp.max(l2, axis=1, keepdims=True)
    i2 = jnp.min(jnp.where(l2 == m2, iota_a, _A), axis=1, keepdims=True)
    selmat = jnp.logical_or(iota_a == i1, iota_a == i2).astype(jnp.float32)
    sel = [selmat[:, a : a + 1] for a in range(_A)]
    xb = x.astype(jnp.bfloat16)
    h = jnp.dot(xb, w1t_ref[...], preferred_element_type=jnp.float32)
    h = h + b1_ref[...]
    h = 0.5 * h * (1.0 + jax.lax.erf(h * 0.7071067811865476))
    hm = jnp.concatenate(
        [h[:, a * _R : (a + 1) * _R] * sel[a] for a in range(_A)], axis=1
    ).astype(jnp.bfloat16)
    y = jnp.dot(hm, w2f_ref[...], preferred_element_type=jnp.float32)
    bias = jnp.dot(selmat, b2_ref[...], preferred_element_type=jnp.float32)
    out_ref[...] = y + bias


@jax.jit
def _fused(x2d, wgt, bg2, w1t, b12, w2f, b2):
    grid = (_N // _BLK,)
    return pl.pallas_call(
        _fused_body,
        grid=grid,
        in_specs=[
            pl.BlockSpec((_BLK, _H), lambda i: (i, 0)),
            pl.BlockSpec((_H, _A), lambda i: (0, 0)),
            pl.BlockSpec((1, _A), lambda i: (0, 0)),
            pl.BlockSpec((_H, _A * _R), lambda i: (0, 0)),
            pl.BlockSpec((1, _A * _R), lambda i: (0, 0)),
            pl.BlockSpec((_A * _R, _H), lambda i: (0, 0)),
            pl.BlockSpec((_A, _H), lambda i: (0, 0)),
        ],
        out_specs=pl.BlockSpec((_BLK, _H), lambda i: (i, 0)),
        out_shape=jax.ShapeDtypeStruct((_N, _H), jnp.float32),
        compiler_params=pltpu.CompilerParams(
            dimension_semantics=("arbitrary",),
        ),
    )(x2d, wgt, bg2, w1t, b12, w2f, b2)


def kernel(input_tensor, W1, b1, W2, b2, Wg, bg):
    x2d = input_tensor.reshape(_N, _H)
    wgt = Wg.T
    bg2 = bg.reshape(1, _A)
    w1t = W1.T.astype(jnp.bfloat16)
    b12 = b1.reshape(1, _A * _R)
    scale = 2.0 / _K
    w2f = (W2.transpose(0, 2, 1).reshape(_A * _R, _H) * scale).astype(jnp.bfloat16)
    y = _fused(x2d, wgt, bg2, w1t, b12, w2f, b2 * scale)
    return y.reshape(_B, _L, _H)


# speedup vs baseline: 9.0338x; 1.0817x over previous
import functools

import jax
import jax.numpy as jnp
from jax.experimental import pallas as pl
from jax.experimental.pallas import tpu as pltpu

_B, _L, _H = 2, 2048, 1024
_A, _R, _K = 8, 128, 2
_N = _B * _L
_BLK = 512


def _fused_body(x_ref, wgt_ref, bg_ref, w1t_ref, b1_ref, w2f_ref, b2_ref, out_ref):
    x = x_ref[...]
    logits = jnp.dot(x, wgt_ref[...], preferred_element_type=jnp.float32)
    logits = logits + bg_ref[...]
    iota_a = jax.lax.broadcasted_iota(jnp.int32, (_BLK, _A), 1)
    m1 = jnp.max(logits, axis=1, keepdims=True)
    i1 = jnp.min(jnp.where(logits == m1, iota_a, _A), axis=1, keepdims=True)
    l2 = jnp.where(iota_a == i1, -jnp.inf, logits)
    m2 = jnp.max(l2, axis=1, keepdims=True)
    i2 = jnp.min(jnp.where(l2 == m2, iota_a, _A), axis=1, keepdims=True)
    selmat = jnp.logical_or(iota_a == i1, iota_a == i2).astype(jnp.float32)
    sel = [selmat[:, a : a + 1] for a in range(_A)]
    xb = x.astype(jnp.bfloat16)
    h = jax.lax.dot_general(
        xb, w1t_ref[...],
        dimension_numbers=(((1,), (1,)), ((), ())),
        preferred_element_type=jnp.float32,
    )
    h = h + b1_ref[...]
    h = 0.5 * h * (1.0 + jax.lax.erf(h * 0.7071067811865476))
    hm = jnp.concatenate(
        [h[:, a * _R : (a + 1) * _R] * sel[a] for a in range(_A)], axis=1
    ).astype(jnp.bfloat16)
    y = jnp.dot(hm, w2f_ref[...], preferred_element_type=jnp.float32)
    bias = jnp.dot(selmat, b2_ref[...], preferred_element_type=jnp.float32)
    out_ref[...] = y + bias


@jax.jit
def _fused(x2d, wgt, bg2, w1t, b12, w2f, b2):
    grid = (_N // _BLK,)
    return pl.pallas_call(
        _fused_body,
        grid=grid,
        in_specs=[
            pl.BlockSpec((_BLK, _H), lambda i: (i, 0)),
            pl.BlockSpec((_H, _A), lambda i: (0, 0)),
            pl.BlockSpec((1, _A), lambda i: (0, 0)),
            pl.BlockSpec((_A * _R, _H), lambda i: (0, 0)),
            pl.BlockSpec((1, _A * _R), lambda i: (0, 0)),
            pl.BlockSpec((_A * _R, _H), lambda i: (0, 0)),
            pl.BlockSpec((_A, _H), lambda i: (0, 0)),
        ],
        out_specs=pl.BlockSpec((_BLK, _H), lambda i: (i, 0)),
        out_shape=jax.ShapeDtypeStruct((_N, _H), jnp.float32),
        compiler_params=pltpu.CompilerParams(
            dimension_semantics=("arbitrary",),
        ),
    )(x2d, wgt, bg2, w1t, b12, w2f, b2)


def kernel(input_tensor, W1, b1, W2, b2, Wg, bg):
    x2d = input_tensor.reshape(_N, _H)
    wgt = Wg.T
    bg2 = bg.reshape(1, _A)
    w1t = W1.astype(jnp.bfloat16)
    b12 = b1.reshape(1, _A * _R)
    scale = 2.0 / _K
    w2f = (W2.transpose(0, 2, 1).reshape(_A * _R, _H) * scale).astype(jnp.bfloat16)
    y = _fused(x2d, wgt, bg2, w1t, b12, w2f, b2 * scale)
    return y.reshape(_B, _L, _H)
